# baseline (speedup 1.0000x reference)
import functools

import jax
import jax.numpy as jnp
from jax.experimental import pallas as pl
from jax.experimental.pallas import tpu as pltpu

T, X, N, W, R, Y = 32, 512, 1024, 64, 4, 128
IF = R * W + 3 * W + 5 * R + 3

_F32 = jnp.float32
_HI = jax.lax.Precision.HIGHEST
_LO = jax.lax.Precision.DEFAULT


def _dot(a, b, dn, prec=_HI):
    return jax.lax.dot_general(a, b, dimension_numbers=(dn, ((), ())),
                               preferred_element_type=_F32, precision=prec)


def _oneplus(x):
    return 1.0 + jnp.maximum(x, 0.0) + jnp.log1p(jnp.exp(-jnp.abs(x)))


def _softmax_cols(x):
    m = jnp.max(x, axis=0, keepdims=True)
    e = jnp.exp(x - m)
    return e / jnp.sum(e, axis=0, keepdims=True)


def _split_dot(a, b_hi, b_lo, dn):
    a_hi = a.astype(jnp.bfloat16)
    a_lo = (a - a_hi.astype(_F32)).astype(jnp.bfloat16)
    return (_dot(a_hi, b_hi, dn, _LO) + _dot(a_hi, b_lo, dn, _LO) +
            _dot(a_lo, b_hi, dn, _LO))


def _lstm_gates(z, c):
    ii = z[:, 0 * X:1 * X]
    ff = z[:, 1 * X:2 * X]
    gg = z[:, 2 * X:3 * X]
    oo = z[:, 3 * X:4 * X]
    c = jax.nn.sigmoid(ff) * c + jax.nn.sigmoid(ii) * jnp.tanh(gg)
    h = jax.nn.sigmoid(oo) * jnp.tanh(c)
    return h, c


def _dnc_kernel(x_ref, wx_ref, wh_hi_ref, wh_lo_ref, b_ref,
                im_hi_ref, im_lo_ref, vm_hi_ref, vm_lo_ref, rm_ref,
                out_ref,
                xw_ref, h_ref, c_ref, m_ref, u_ref, ww_ref, l_ref, p_ref,
                rw_ref, mn_ref, iv_ref):
    i = pl.program_id(0)

    @pl.when(i == 0)
    def _init():
        xw = _dot(x_ref[...], wx_ref[...], ((1,), (0,))) + b_ref[...]
        xw_ref[...] = xw
        h0, c0 = _lstm_gates(xw[0:1, :], jnp.zeros((1, X), _F32))
        h_ref[...] = h0
        c_ref[...] = c0
        iv_ref[...] = _split_dot(h0, im_hi_ref[...], im_lo_ref[...],
                                 ((1,), (0,)))
        m_ref[...] = jnp.zeros_like(m_ref)
        u_ref[...] = jnp.zeros_like(u_ref)
        ww_ref[...] = jnp.zeros_like(ww_ref)
        l_ref[...] = jnp.zeros_like(l_ref)
        p_ref[...] = jnp.zeros_like(p_ref)
        rw_ref[...] = jnp.zeros_like(rw_ref)
        mn_ref[...] = jnp.zeros_like(mn_ref)

    h = h_ref[...]
    c = c_ref[...]
    iv = iv_ref[...]

    zn = (xw_ref[pl.ds(jnp.minimum(i + 1, T - 1), 1), :] +
          _split_dot(h, wh_hi_ref[...], wh_lo_ref[...], ((1,), (0,))))
    h_next, c_next = _lstm_gates(zn, c)
    iv_next = _split_dot(h_next, im_hi_ref[...], im_lo_ref[...],
                         ((1,), (0,)))
    h_ref[...] = h_next
    c_ref[...] = c_next
    iv_ref[...] = iv_next

    p = 0
    read_keys = iv[:, p:p + R * W]; p += R * W
    read_strengths = _oneplus(iv[:, p:p + R]); p += R
    write_key = iv[:, p:p + W]; p += W
    write_strength = _oneplus(iv[:, p:p + 1]); p += 1
    erase = jax.nn.sigmoid(iv[:, p:p + W]); p += W
    write_vec = iv[:, p:p + W]; p += W
    free_gates = jax.nn.sigmoid(iv[:, p:p + R]); p += R
    alloc_gate = jax.nn.sigmoid(iv[:, p:p + 1]); p += 1
    write_gate = jax.nn.sigmoid(iv[:, p:p + 1]); p += 1
    rm_base = p

    rw_old = rw_ref[...]
    usage = u_ref[...]
    ww_old = ww_ref[...]

    ret_terms = 1.0 - free_gates * rw_old
    retention = (ret_terms[:, 0:1] * ret_terms[:, 1:2] *
                 ret_terms[:, 2:3] * ret_terms[:, 3:4])
    u = retention * (usage + ww_old - usage * ww_old)

    idx_col = jax.lax.broadcasted_iota(jnp.int32, (N, 1), 0)
    idx_row = jax.lax.broadcasted_iota(jnp.int32, (1, N), 1)
    u_row = jnp.transpose(u)
    log_u_row = jnp.where(u_row > 0.0, jnp.log(u_row), -1e5)
    log_prod = jnp.sum(
        jnp.where((u_row < u) | ((u_row == u) & (idx_row < idx_col)),
                  log_u_row, 0.0),
        axis=1, keepdims=True)
    alloc = (1.0 - u) * jnp.exp(log_prod)

    m = m_ref[...]
    row_norm = mn_ref[...]
    wk_norm = jnp.sqrt(jnp.sum(write_key * write_key, axis=1, keepdims=True))
    mk = _dot(m, write_key, ((1,), (1,)), _LO)
    wcw = _softmax_cols(write_strength * mk / (row_norm * wk_norm + 1e-8))

    ww = write_gate * (alloc_gate * alloc + (1.0 - alloc_gate) * wcw)

    m = m * (1.0 - ww * erase) + ww * write_vec

    lold = l_ref[...]
    prec = p_ref[...] * (1.0 - jnp.sum(ww)) + ww
    ww_row = jnp.transpose(ww)
    prec_row = jnp.transpose(prec)
    lnk = lold * (1.0 - ww - ww_row) + ww * prec_row

    rwcat = jnp.concatenate([rw_old, rw_old * ww], axis=1)
    fz = _dot(lold, rwcat, ((0,), (0,)), _LO)
    bz = _dot(lold, rwcat, ((1,), (0,)), _LO)
    s_ww = jnp.sum(rw_old * ww, axis=0, keepdims=True)
    s_pr = jnp.sum(rw_old * prec, axis=0, keepdims=True)
    fw = fz[:, :R] - fz[:, R:] - fz[:, :R] * ww + prec * s_ww
    bw = bz[:, :R] - bz[:, R:] - bz[:, :R] * ww + ww * s_pr

    keys = jnp.concatenate(
        [read_keys[:, r * W:(r + 1) * W] for r in range(R)], axis=0)
    k_norm = jnp.sqrt(jnp.sum(keys * keys, axis=1, keepdims=True))
    row_norm2 = jnp.sqrt(jnp.sum(m * m, axis=1, keepdims=True))
    mkr = _dot(m, keys, ((1,), (1,)), _LO)
    cos = mkr / (row_norm2 * jnp.transpose(k_norm) + 1e-8)
    rcw = _softmax_cols(read_strengths * cos)

    mode_b, mode_c, mode_f = [], [], []
    for r in range(R):
        lg = iv[:, rm_base + 3 * r:rm_base + 3 * (r + 1)]
        e = jnp.exp(lg - jnp.max(lg, axis=1, keepdims=True))
        sm = e / jnp.sum(e, axis=1, keepdims=True)
        mode_b.append(sm[:, 0:1])
        mode_c.append(sm[:, 1:2])
        mode_f.append(sm[:, 2:3])
    mode_b = jnp.concatenate(mode_b, axis=1)
    mode_c = jnp.concatenate(mode_c, axis=1)
    mode_f = jnp.concatenate(mode_f, axis=1)

    rw = mode_b * bw + mode_c * rcw + mode_f * fw
    reads = _dot(rw, m, ((0,), (0,)), _LO)

    out = _split_dot(h, vm_hi_ref[...], vm_lo_ref[...], ((1,), (0,)))
    for r in range(R):
        out = out + _dot(reads[r:r + 1, :], rm_ref[r * W:(r + 1) * W, :],
                         ((1,), (0,)))
    out_ref[...] = out[None]

    m_ref[...] = m
    u_ref[...] = u
    ww_ref[...] = ww
    l_ref[...] = lnk
    p_ref[...] = prec
    rw_ref[...] = rw
    mn_ref[...] = row_norm2


@jax.jit
def kernel(inputs, W_lstm, b_lstm, IM, VM, RM):
    wx = W_lstm[:X]
    wh = W_lstm[X:]
    wh_hi = wh.astype(jnp.bfloat16)
    wh_lo = (wh - wh_hi.astype(_F32)).astype(jnp.bfloat16)
    b = b_lstm.reshape(1, 4 * X)
    im_p = jnp.pad(IM, ((0, 0), (0, 512 - IF)))
    im_hi = im_p.astype(jnp.bfloat16)
    im_lo = (im_p - im_hi.astype(_F32)).astype(jnp.bfloat16)
    vm_hi = VM.astype(jnp.bfloat16)
    vm_lo = (VM - vm_hi.astype(_F32)).astype(jnp.bfloat16)

    out = pl.pallas_call(
        _dnc_kernel,
        grid=(T,),
        in_specs=[
            pl.BlockSpec((T, X), lambda i: (0, 0)),
            pl.BlockSpec((X, 4 * X), lambda i: (0, 0)),
            pl.BlockSpec((X, 4 * X), lambda i: (0, 0)),
            pl.BlockSpec((X, 4 * X), lambda i: (0, 0)),
            pl.BlockSpec((1, 4 * X), lambda i: (0, 0)),
            pl.BlockSpec((X, 512), lambda i: (0, 0)),
            pl.BlockSpec((X, 512), lambda i: (0, 0)),
            pl.BlockSpec((X, Y), lambda i: (0, 0)),
            pl.BlockSpec((X, Y), lambda i: (0, 0)),
            pl.BlockSpec((R * W, Y), lambda i: (0, 0)),
        ],
        out_specs=pl.BlockSpec((1, 1, Y), lambda i: (i, 0, 0)),
        out_shape=jax.ShapeDtypeStruct((T, 1, Y), _F32),
        scratch_shapes=[
            pltpu.VMEM((T, 4 * X), _F32),
            pltpu.VMEM((1, X), _F32),
            pltpu.VMEM((1, X), _F32),
            pltpu.VMEM((N, W), _F32),
            pltpu.VMEM((N, 1), _F32),
            pltpu.VMEM((N, 1), _F32),
            pltpu.VMEM((N, N), _F32),
            pltpu.VMEM((N, 1), _F32),
            pltpu.VMEM((N, R), _F32),
            pltpu.VMEM((N, 1), _F32),
            pltpu.VMEM((1, 512), _F32),
        ],
    )(inputs, wx, wh_hi, wh_lo, b, im_hi, im_lo, vm_hi, vm_lo, RM)
    return out.reshape(T, Y)


# speedup vs baseline: 1.0487x; 1.0487x over previous
import functools

import jax
import jax.numpy as jnp
from jax.experimental import pallas as pl
from jax.experimental.pallas import tpu as pltpu

T, X, N, W, R, Y = 32, 512, 1024, 64, 4, 128
IF = R * W + 3 * W + 5 * R + 3

_F32 = jnp.float32
_HI = jax.lax.Precision.HIGHEST
_LO = jax.lax.Precision.DEFAULT


def _dot(a, b, dn, prec=_HI):
    return jax.lax.dot_general(a, b, dimension_numbers=(dn, ((), ())),
                               preferred_element_type=_F32, precision=prec)


def _oneplus(x):
    return 1.0 + jnp.maximum(x, 0.0) + jnp.log1p(jnp.exp(-jnp.abs(x)))


def _softmax_cols(x):
    m = jnp.max(x, axis=0, keepdims=True)
    e = jnp.exp(x - m)
    return e / jnp.sum(e, axis=0, keepdims=True)


def _split_dot(a, b_hi, b_lo, dn):
    a_hi = a.astype(jnp.bfloat16)
    a_lo = (a - a_hi.astype(_F32)).astype(jnp.bfloat16)
    return (_dot(a_hi, b_hi, dn, _LO) + _dot(a_hi, b_lo, dn, _LO) +
            _dot(a_lo, b_hi, dn, _LO))


def _lstm_gates(z, c):
    ii = z[:, 0 * X:1 * X]
    ff = z[:, 1 * X:2 * X]
    gg = z[:, 2 * X:3 * X]
    oo = z[:, 3 * X:4 * X]
    c = jax.nn.sigmoid(ff) * c + jax.nn.sigmoid(ii) * jnp.tanh(gg)
    h = jax.nn.sigmoid(oo) * jnp.tanh(c)
    return h, c


def _dnc_kernel(x_ref, wx_ref, wh_hi_ref, wh_lo_ref, b_ref,
                im_hi_ref, im_lo_ref, vm_hi_ref, vm_lo_ref, rm_ref,
                out_ref,
                xw_ref, h_ref, c_ref, m_ref, u_ref, ww_ref, l_ref, p_ref,
                rw_ref, mn_ref, iv_ref):
    i = pl.program_id(0)

    @pl.when(i == 0)
    def _init():
        xw = _dot(x_ref[...], wx_ref[...], ((1,), (0,))) + b_ref[...]
        xw_ref[...] = xw
        h0, c0 = _lstm_gates(xw[0:1, :], jnp.zeros((1, X), _F32))
        h_ref[...] = h0
        c_ref[...] = c0
        iv_ref[...] = _split_dot(h0, im_hi_ref[...], im_lo_ref[...],
                                 ((1,), (0,)))
        m_ref[...] = jnp.zeros_like(m_ref)
        u_ref[...] = jnp.zeros_like(u_ref)
        ww_ref[...] = jnp.zeros_like(ww_ref)
        l_ref[...] = jnp.zeros_like(l_ref)
        p_ref[...] = jnp.zeros_like(p_ref)
        rw_ref[...] = jnp.zeros_like(rw_ref)
        mn_ref[...] = jnp.zeros_like(mn_ref)

    h = h_ref[...]
    c = c_ref[...]
    iv = iv_ref[...]

    zn = (xw_ref[pl.ds(jnp.minimum(i + 1, T - 1), 1), :] +
          _split_dot(h, wh_hi_ref[...], wh_lo_ref[...], ((1,), (0,))))
    h_next, c_next = _lstm_gates(zn, c)
    iv_next = _split_dot(h_next, im_hi_ref[...], im_lo_ref[...],
                         ((1,), (0,)))
    h_ref[...] = h_next
    c_ref[...] = c_next
    iv_ref[...] = iv_next

    p = 0
    read_keys = iv[:, p:p + R * W]; p += R * W
    read_strengths = _oneplus(iv[:, p:p + R]); p += R
    write_key = iv[:, p:p + W]; p += W
    write_strength = _oneplus(iv[:, p:p + 1]); p += 1
    erase = jax.nn.sigmoid(iv[:, p:p + W]); p += W
    write_vec = iv[:, p:p + W]; p += W
    free_gates = jax.nn.sigmoid(iv[:, p:p + R]); p += R
    alloc_gate = jax.nn.sigmoid(iv[:, p:p + 1]); p += 1
    write_gate = jax.nn.sigmoid(iv[:, p:p + 1]); p += 1
    rm_base = p

    rw_old = rw_ref[...]
    usage = u_ref[...]
    ww_old = ww_ref[...]

    ret_terms = 1.0 - free_gates * rw_old
    retention = (ret_terms[:, 0:1] * ret_terms[:, 1:2] *
                 ret_terms[:, 2:3] * ret_terms[:, 3:4])
    u = retention * (usage + ww_old - usage * ww_old)

    idx_col = jax.lax.broadcasted_iota(jnp.int32, (N, 1), 0)
    idx_row = jax.lax.broadcasted_iota(jnp.int32, (1, N), 1)
    u_row = jnp.transpose(u)
    log_u_row = jnp.where(u_row > 0.0, jnp.log(u_row), -1e5)
    log_prod = jnp.sum(
        jnp.where((u_row < u) | ((u_row == u) & (idx_row < idx_col)),
                  log_u_row, 0.0),
        axis=1, keepdims=True)
    alloc = (1.0 - u) * jnp.exp(log_prod)

    m = m_ref[...]
    row_norm = mn_ref[...]
    wk_norm = jnp.sqrt(jnp.sum(write_key * write_key, axis=1, keepdims=True))
    mk = _dot(m, write_key, ((1,), (1,)), _LO)
    wcw = _softmax_cols(write_strength * mk / (row_norm * wk_norm + 1e-8))

    ww = write_gate * (alloc_gate * alloc + (1.0 - alloc_gate) * wcw)

    m = m * (1.0 - ww * erase) + ww * write_vec

    lold = l_ref[...]
    prec = p_ref[...] * (1.0 - jnp.sum(ww)) + ww
    ww_row = jnp.transpose(ww)
    prec_row = jnp.transpose(prec)
    lnk = lold * (1.0 - ww - ww_row) + ww * prec_row

    rww = rw_old * ww
    fz0 = _dot(lold, rw_old, ((0,), (0,)), _LO)
    bz0 = _dot(lold, rw_old, ((1,), (0,)), _LO)
    fzw = _dot(lold, rww, ((0,), (0,)), _LO)
    bzw = _dot(lold, rww, ((1,), (0,)), _LO)
    s_ww = jnp.sum(rww, axis=0, keepdims=True)
    s_pr = jnp.sum(rw_old * prec, axis=0, keepdims=True)
    fw = fz0 - fzw - fz0 * ww + prec * s_ww
    bw = bz0 - bzw - bz0 * ww + ww * s_pr

    keys = jnp.concatenate(
        [read_keys[:, r * W:(r + 1) * W] for r in range(R)], axis=0)
    k_norm = jnp.sqrt(jnp.sum(keys * keys, axis=1, keepdims=True))
    row_norm2 = jnp.sqrt(jnp.sum(m * m, axis=1, keepdims=True))
    mkr = _dot(m, keys, ((1,), (1,)), _LO)
    cos = mkr / (row_norm2 * jnp.transpose(k_norm) + 1e-8)
    rcw = _softmax_cols(read_strengths * cos)

    mode_b, mode_c, mode_f = [], [], []
    for r in range(R):
        lg = iv[:, rm_base + 3 * r:rm_base + 3 * (r + 1)]
        e = jnp.exp(lg - jnp.max(lg, axis=1, keepdims=True))
        sm = e / jnp.sum(e, axis=1, keepdims=True)
        mode_b.append(sm[:, 0:1])
        mode_c.append(sm[:, 1:2])
        mode_f.append(sm[:, 2:3])
    mode_b = jnp.concatenate(mode_b, axis=1)
    mode_c = jnp.concatenate(mode_c, axis=1)
    mode_f = jnp.concatenate(mode_f, axis=1)

    rw = mode_b * bw + mode_c * rcw + mode_f * fw
    reads = _dot(rw, m, ((0,), (0,)), _LO)

    out = _split_dot(h, vm_hi_ref[...], vm_lo_ref[...], ((1,), (0,)))
    for r in range(R):
        out = out + _dot(reads[r:r + 1, :], rm_ref[r * W:(r + 1) * W, :],
                         ((1,), (0,)))
    out_ref[...] = out[None]

    m_ref[...] = m
    u_ref[...] = u
    ww_ref[...] = ww
    l_ref[...] = lnk
    p_ref[...] = prec
    rw_ref[...] = rw
    mn_ref[...] = row_norm2


@jax.jit
def kernel(inputs, W_lstm, b_lstm, IM, VM, RM):
    wx = W_lstm[:X]
    wh = W_lstm[X:]
    wh_hi = wh.astype(jnp.bfloat16)
    wh_lo = (wh - wh_hi.astype(_F32)).astype(jnp.bfloat16)
    b = b_lstm.reshape(1, 4 * X)
    im_p = jnp.pad(IM, ((0, 0), (0, 512 - IF)))
    im_hi = im_p.astype(jnp.bfloat16)
    im_lo = (im_p - im_hi.astype(_F32)).astype(jnp.bfloat16)
    vm_hi = VM.astype(jnp.bfloat16)
    vm_lo = (VM - vm_hi.astype(_F32)).astype(jnp.bfloat16)

    out = pl.pallas_call(
        _dnc_kernel,
        grid=(T,),
        in_specs=[
            pl.BlockSpec((T, X), lambda i: (0, 0)),
            pl.BlockSpec((X, 4 * X), lambda i: (0, 0)),
            pl.BlockSpec((X, 4 * X), lambda i: (0, 0)),
            pl.BlockSpec((X, 4 * X), lambda i: (0, 0)),
            pl.BlockSpec((1, 4 * X), lambda i: (0, 0)),
            pl.BlockSpec((X, 512), lambda i: (0, 0)),
            pl.BlockSpec((X, 512), lambda i: (0, 0)),
            pl.BlockSpec((X, Y), lambda i: (0, 0)),
            pl.BlockSpec((X, Y), lambda i: (0, 0)),
            pl.BlockSpec((R * W, Y), lambda i: (0, 0)),
        ],
        out_specs=pl.BlockSpec((1, 1, Y), lambda i: (i, 0, 0)),
        out_shape=jax.ShapeDtypeStruct((T, 1, Y), _F32),
        scratch_shapes=[
            pltpu.VMEM((T, 4 * X), _F32),
            pltpu.VMEM((1, X), _F32),
            pltpu.VMEM((1, X), _F32),
            pltpu.VMEM((N, W), _F32),
            pltpu.VMEM((N, 1), _F32),
            pltpu.VMEM((N, 1), _F32),
            pltpu.VMEM((N, N), _F32),
            pltpu.VMEM((N, 1), _F32),
            pltpu.VMEM((N, R), _F32),
            pltpu.VMEM((N, 1), _F32),
            pltpu.VMEM((1, 512), _F32),
        ],
    )(inputs, wx, wh_hi, wh_lo, b, im_hi, im_lo, vm_hi, vm_lo, RM)
    return out.reshape(T, Y)


# speedup vs baseline: 1.2539x; 1.1957x over previous
import functools

import jax
import jax.numpy as jnp
from jax.experimental import pallas as pl
from jax.experimental.pallas import tpu as pltpu

T, X, N, W, R, Y = 32, 512, 1024, 64, 4, 128
IF = R * W + 3 * W + 5 * R + 3

_F32 = jnp.float32
_HI = jax.lax.Precision.HIGHEST
_LO = jax.lax.Precision.DEFAULT


def _dot(a, b, dn, prec=_HI):
    return jax.lax.dot_general(a, b, dimension_numbers=(dn, ((), ())),
                               preferred_element_type=_F32, precision=prec)


def _oneplus(x):
    return 1.0 + jnp.maximum(x, 0.0) + jnp.log1p(jnp.exp(-jnp.abs(x)))


def _softmax_cols(x):
    m = jnp.max(x, axis=0, keepdims=True)
    e = jnp.exp(x - m)
    return e / jnp.sum(e, axis=0, keepdims=True)


def _split_dot(a, b_hi, b_lo, dn):
    a_hi = a.astype(jnp.bfloat16)
    a_lo = (a - a_hi.astype(_F32)).astype(jnp.bfloat16)
    return (_dot(a_hi, b_hi, dn, _LO) + _dot(a_hi, b_lo, dn, _LO) +
            _dot(a_lo, b_hi, dn, _LO))


def _lstm_gates(z, c):
    ii = z[:, 0 * X:1 * X]
    ff = z[:, 1 * X:2 * X]
    gg = z[:, 2 * X:3 * X]
    oo = z[:, 3 * X:4 * X]
    c = jax.nn.sigmoid(ff) * c + jax.nn.sigmoid(ii) * jnp.tanh(gg)
    h = jax.nn.sigmoid(oo) * jnp.tanh(c)
    return h, c


def _dnc_kernel(x_ref, wx_ref, wh_hi_ref, wh_lo_ref, b_ref,
                im_hi_ref, im_lo_ref, vm_hi_ref, vm_lo_ref, rm_ref,
                out_ref,
                xw_ref, h_ref, c_ref, m_ref, u_ref, ww_ref, l_ref, p_ref,
                rw_ref, mn_ref, iv_ref):
    i = pl.program_id(0)

    @pl.when(i == 0)
    def _init():
        xw = _dot(x_ref[...], wx_ref[...], ((1,), (0,))) + b_ref[...]
        xw_ref[...] = xw
        h0, c0 = _lstm_gates(xw[0:1, :], jnp.zeros((1, X), _F32))
        h_ref[...] = h0
        c_ref[...] = c0
        iv_ref[...] = _split_dot(h0, im_hi_ref[...], im_lo_ref[...],
                                 ((1,), (0,)))
        m_ref[...] = jnp.zeros_like(m_ref)
        u_ref[...] = jnp.zeros_like(u_ref)
        ww_ref[...] = jnp.zeros_like(ww_ref)
        l_ref[...] = jnp.zeros_like(l_ref)
        p_ref[...] = jnp.zeros_like(p_ref)
        rw_ref[...] = jnp.zeros_like(rw_ref)
        mn_ref[...] = jnp.zeros_like(mn_ref)

    h = h_ref[...]
    c = c_ref[...]
    iv = iv_ref[...]

    zn = (xw_ref[pl.ds(jnp.minimum(i + 1, T - 1), 1), :] +
          _split_dot(h, wh_hi_ref[...], wh_lo_ref[...], ((1,), (0,))))
    h_next, c_next = _lstm_gates(zn, c)
    iv_next = _split_dot(h_next, im_hi_ref[...], im_lo_ref[...],
                         ((1,), (0,)))
    h_ref[...] = h_next
    c_ref[...] = c_next
    iv_ref[...] = iv_next

    p = 0
    read_keys = iv[:, p:p + R * W]; p += R * W
    read_strengths = _oneplus(iv[:, p:p + R]); p += R
    write_key = iv[:, p:p + W]; p += W
    write_strength = _oneplus(iv[:, p:p + 1]); p += 1
    erase = jax.nn.sigmoid(iv[:, p:p + W]); p += W
    write_vec = iv[:, p:p + W]; p += W
    free_gates = jax.nn.sigmoid(iv[:, p:p + R]); p += R
    alloc_gate = jax.nn.sigmoid(iv[:, p:p + 1]); p += 1
    write_gate = jax.nn.sigmoid(iv[:, p:p + 1]); p += 1
    rm_base = p

    rw_old = rw_ref[...]
    usage = u_ref[...]
    ww_old = ww_ref[...]

    fg_col = jnp.transpose(free_gates)
    ret_terms = 1.0 - fg_col * rw_old
    retention = (ret_terms[0:1] * ret_terms[1:2] *
                 ret_terms[2:3] * ret_terms[3:4])
    u = retention * (usage + ww_old - usage * ww_old)

    idx_col = jax.lax.broadcasted_iota(jnp.int32, (N, 1), 0)
    idx_row = jax.lax.broadcasted_iota(jnp.int32, (1, N), 1)
    u_col = jnp.transpose(u)
    log_u = jnp.where(u > 0.0, jnp.log(u), -1e5)
    log_u_col = jnp.transpose(log_u)
    log_prod = jnp.sum(
        jnp.where((u_col < u) | ((u_col == u) & (idx_col < idx_row)),
                  log_u_col, 0.0),
        axis=0, keepdims=True)
    alloc = (1.0 - u) * jnp.exp(log_prod)

    m = m_ref[...]
    row_norm = mn_ref[...]
    wk_norm = jnp.sqrt(jnp.sum(write_key * write_key, axis=1, keepdims=True))
    mk = _dot(write_key, m, ((1,), (1,)), _LO)
    wlog = write_strength * mk / (row_norm * wk_norm + 1e-8)
    wcw = jax.nn.softmax(wlog, axis=1)

    ww = write_gate * (alloc_gate * alloc + (1.0 - alloc_gate) * wcw)

    ww_col = jnp.transpose(ww)
    m = m * (1.0 - ww_col * erase) + ww_col * write_vec

    prec = p_ref[...] * (1.0 - jnp.sum(ww)) + ww
    lnk = l_ref[...] * (1.0 - ww_col - ww) + ww_col * prec

    fw = _dot(rw_old, lnk, ((1,), (0,)), _LO)
    bw = _dot(rw_old, lnk, ((1,), (1,)), _LO)

    keys = jnp.concatenate(
        [read_keys[:, r * W:(r + 1) * W] for r in range(R)], axis=0)
    k_norm = jnp.sqrt(jnp.sum(keys * keys, axis=1, keepdims=True))
    row_norm2 = jnp.sqrt(_dot(jnp.ones((1, W), _F32), m * m,
                              ((1,), (1,)), _LO))
    mkr = _dot(keys, m, ((1,), (1,)), _LO)
    cos = mkr / (k_norm * row_norm2 + 1e-8)
    rs_col = jnp.transpose(read_strengths)
    rcw = jax.nn.softmax(rs_col * cos, axis=1)

    mode_b, mode_c, mode_f = [], [], []
    for r in range(R):
        lg = iv[:, rm_base + 3 * r:rm_base + 3 * (r + 1)]
        e = jnp.exp(lg - jnp.max(lg, axis=1, keepdims=True))
        sm = e / jnp.sum(e, axis=1, keepdims=True)
        mode_b.append(sm[:, 0:1])
        mode_c.append(sm[:, 1:2])
        mode_f.append(sm[:, 2:3])
    mode_b = jnp.concatenate(mode_b, axis=0)
    mode_c = jnp.concatenate(mode_c, axis=0)
    mode_f = jnp.concatenate(mode_f, axis=0)

    rw = mode_b * bw + mode_c * rcw + mode_f * fw
    reads = _dot(rw, m, ((1,), (0,)), _LO)

    out = _split_dot(h, vm_hi_ref[...], vm_lo_ref[...], ((1,), (0,)))
    for r in range(R):
        out = out + _dot(reads[r:r + 1, :], rm_ref[r * W:(r + 1) * W, :],
                         ((1,), (0,)))
    out_ref[...] = out[None]

    m_ref[...] = m
    u_ref[...] = u
    ww_ref[...] = ww
    l_ref[...] = lnk
    p_ref[...] = prec
    rw_ref[...] = rw
    mn_ref[...] = row_norm2


@jax.jit
def kernel(inputs, W_lstm, b_lstm, IM, VM, RM):
    wx = W_lstm[:X]
    wh = W_lstm[X:]
    wh_hi = wh.astype(jnp.bfloat16)
    wh_lo = (wh - wh_hi.astype(_F32)).astype(jnp.bfloat16)
    b = b_lstm.reshape(1, 4 * X)
    im_p = jnp.pad(IM, ((0, 0), (0, 512 - IF)))
    im_hi = im_p.astype(jnp.bfloat16)
    im_lo = (im_p - im_hi.astype(_F32)).astype(jnp.bfloat16)
    vm_hi = VM.astype(jnp.bfloat16)
    vm_lo = (VM - vm_hi.astype(_F32)).astype(jnp.bfloat16)

    out = pl.pallas_call(
        _dnc_kernel,
        grid=(T,),
        in_specs=[
            pl.BlockSpec((T, X), lambda i: (0, 0)),
            pl.BlockSpec((X, 4 * X), lambda i: (0, 0)),
            pl.BlockSpec((X, 4 * X), lambda i: (0, 0)),
            pl.BlockSpec((X, 4 * X), lambda i: (0, 0)),
            pl.BlockSpec((1, 4 * X), lambda i: (0, 0)),
            pl.BlockSpec((X, 512), lambda i: (0, 0)),
            pl.BlockSpec((X, 512), lambda i: (0, 0)),
            pl.BlockSpec((X, Y), lambda i: (0, 0)),
            pl.BlockSpec((X, Y), lambda i: (0, 0)),
            pl.BlockSpec((R * W, Y), lambda i: (0, 0)),
        ],
        out_specs=pl.BlockSpec((1, 1, Y), lambda i: (i, 0, 0)),
        out_shape=jax.ShapeDtypeStruct((T, 1, Y), _F32),
        scratch_shapes=[
            pltpu.VMEM((T, 4 * X), _F32),
            pltpu.VMEM((1, X), _F32),
            pltpu.VMEM((1, X), _F32),
            pltpu.VMEM((N, W), _F32),
            pltpu.VMEM((1, N), _F32),
            pltpu.VMEM((1, N), _F32),
            pltpu.VMEM((N, N), _F32),
            pltpu.VMEM((1, N), _F32),
            pltpu.VMEM((R, N), _F32),
            pltpu.VMEM((1, N), _F32),
            pltpu.VMEM((1, 512), _F32),
        ],
    )(inputs, wx, wh_hi, wh_lo, b, im_hi, im_lo, vm_hi, vm_lo, RM)
    return out.reshape(T, Y)


# speedup vs baseline: 1.2955x; 1.0332x over previous
import functools

import jax
import jax.numpy as jnp
from jax.experimental import pallas as pl
from jax.experimental.pallas import tpu as pltpu

T, X, N, W, R, Y = 32, 512, 1024, 64, 4, 128
IF = R * W + 3 * W + 5 * R + 3

_F32 = jnp.float32
_HI = jax.lax.Precision.HIGHEST
_LO = jax.lax.Precision.DEFAULT


def _dot(a, b, dn, prec=_HI):
    return jax.lax.dot_general(a, b, dimension_numbers=(dn, ((), ())),
                               preferred_element_type=_F32, precision=prec)


def _oneplus(x):
    return 1.0 + jnp.maximum(x, 0.0) + jnp.log1p(jnp.exp(-jnp.abs(x)))


def _softmax_cols(x):
    m = jnp.max(x, axis=0, keepdims=True)
    e = jnp.exp(x - m)
    return e / jnp.sum(e, axis=0, keepdims=True)


def _split_dot(a, b_hi, b_lo, dn):
    a_hi = a.astype(jnp.bfloat16)
    a_lo = (a - a_hi.astype(_F32)).astype(jnp.bfloat16)
    return (_dot(a_hi, b_hi, dn, _LO) + _dot(a_hi, b_lo, dn, _LO) +
            _dot(a_lo, b_hi, dn, _LO))


def _lstm_gates(z, c):
    ii = z[:, 0 * X:1 * X]
    ff = z[:, 1 * X:2 * X]
    gg = z[:, 2 * X:3 * X]
    oo = z[:, 3 * X:4 * X]
    c = jax.nn.sigmoid(ff) * c + jax.nn.sigmoid(ii) * jnp.tanh(gg)
    h = jax.nn.sigmoid(oo) * jnp.tanh(c)
    return h, c


def _dnc_kernel(x_ref, wx_ref, wh_hi_ref, wh_lo_ref, b_ref,
                im_hi_ref, im_lo_ref, vm_hi_ref, vm_lo_ref, rm_hi_ref,
                rm_lo_ref,
                out_ref,
                xw_ref, h_ref, c_ref, m_ref, u_ref, ww_ref, l_ref, p_ref,
                rw_ref, mn_ref, iv_ref):
    i = pl.program_id(0)

    @pl.when(i == 0)
    def _init():
        xw = _dot(x_ref[...], wx_ref[...], ((1,), (0,))) + b_ref[...]
        xw_ref[...] = xw
        h0, c0 = _lstm_gates(xw[0:1, :], jnp.zeros((1, X), _F32))
        h_ref[...] = h0
        c_ref[...] = c0
        iv_ref[...] = _split_dot(h0, im_hi_ref[...], im_lo_ref[...],
                                 ((1,), (0,)))
        m_ref[...] = jnp.zeros_like(m_ref)
        u_ref[...] = jnp.zeros_like(u_ref)
        ww_ref[...] = jnp.zeros_like(ww_ref)
        l_ref[...] = jnp.zeros_like(l_ref)
        p_ref[...] = jnp.zeros_like(p_ref)
        rw_ref[...] = jnp.zeros_like(rw_ref)
        mn_ref[...] = jnp.zeros_like(mn_ref)

    h = h_ref[...]
    c = c_ref[...]
    iv = iv_ref[...]

    zn = (xw_ref[pl.ds(jnp.minimum(i + 1, T - 1), 1), :] +
          _split_dot(h, wh_hi_ref[...], wh_lo_ref[...], ((1,), (0,))))
    h_next, c_next = _lstm_gates(zn, c)
    iv_next = _split_dot(h_next, im_hi_ref[...], im_lo_ref[...],
                         ((1,), (0,)))
    h_ref[...] = h_next
    c_ref[...] = c_next
    iv_ref[...] = iv_next

    p = 0
    read_keys = iv[:, p:p + R * W]; p += R * W
    read_strengths = _oneplus(iv[:, p:p + R]); p += R
    write_key = iv[:, p:p + W]; p += W
    write_strength = _oneplus(iv[:, p:p + 1]); p += 1
    erase = jax.nn.sigmoid(iv[:, p:p + W]); p += W
    write_vec = iv[:, p:p + W]; p += W
    free_gates = jax.nn.sigmoid(iv[:, p:p + R]); p += R
    alloc_gate = jax.nn.sigmoid(iv[:, p:p + 1]); p += 1
    write_gate = jax.nn.sigmoid(iv[:, p:p + 1]); p += 1
    rm_base = p

    rw_old = rw_ref[...]
    usage = u_ref[...]
    ww_old = ww_ref[...]

    fg_col = jnp.transpose(free_gates)
    ret_terms = 1.0 - fg_col * rw_old
    retention = (ret_terms[0:1] * ret_terms[1:2] *
                 ret_terms[2:3] * ret_terms[3:4])
    u = retention * (usage + ww_old - usage * ww_old)

    idx_col = jax.lax.broadcasted_iota(jnp.int32, (N, 1), 0)
    idx_row = jax.lax.broadcasted_iota(jnp.int32, (1, N), 1)
    u_col = jnp.transpose(u)
    log_u = jnp.where(u > 0.0, jnp.log(u), -1e5)
    log_u_col = jnp.transpose(log_u)
    log_prod = jnp.sum(
        jnp.where((u_col < u) | ((u_col == u) & (idx_col < idx_row)),
                  log_u_col, 0.0),
        axis=0, keepdims=True)
    alloc = (1.0 - u) * jnp.exp(log_prod)

    m = m_ref[...]
    row_norm = mn_ref[...]
    wk_norm = jnp.sqrt(jnp.sum(write_key * write_key, axis=1, keepdims=True))
    mk = _dot(write_key, m, ((1,), (1,)), _LO)
    wlog = write_strength * mk / (row_norm * wk_norm + 1e-8)
    wcw = jax.nn.softmax(wlog, axis=1)

    ww = write_gate * (alloc_gate * alloc + (1.0 - alloc_gate) * wcw)

    ww_col = jnp.transpose(ww)
    m = m * (1.0 - ww_col * erase) + ww_col * write_vec

    prec = p_ref[...] * (1.0 - jnp.sum(ww)) + ww
    lnk = l_ref[...] * (1.0 - ww_col - ww) + ww_col * prec

    fw = _dot(rw_old, lnk, ((1,), (0,)), _LO)
    bw = _dot(rw_old, lnk, ((1,), (1,)), _LO)

    keys = jnp.concatenate(
        [read_keys[:, r * W:(r + 1) * W] for r in range(R)], axis=0)
    k_norm = jnp.sqrt(jnp.sum(keys * keys, axis=1, keepdims=True))
    row_norm2 = jnp.sqrt(_dot(jnp.ones((1, W), _F32), m * m,
                              ((1,), (1,)), _LO))
    mkr = _dot(keys, m, ((1,), (1,)), _LO)
    cos = mkr / (k_norm * row_norm2 + 1e-8)
    rs_col = jnp.transpose(read_strengths)
    rcw = jax.nn.softmax(rs_col * cos, axis=1)

    mode_b, mode_c, mode_f = [], [], []
    for r in range(R):
        lg = iv[:, rm_base + 3 * r:rm_base + 3 * (r + 1)]
        e = jnp.exp(lg - jnp.max(lg, axis=1, keepdims=True))
        sm = e / jnp.sum(e, axis=1, keepdims=True)
        mode_b.append(sm[:, 0:1])
        mode_c.append(sm[:, 1:2])
        mode_f.append(sm[:, 2:3])
    mode_b = jnp.concatenate(mode_b, axis=0)
    mode_c = jnp.concatenate(mode_c, axis=0)
    mode_f = jnp.concatenate(mode_f, axis=0)

    rw = mode_b * bw + mode_c * rcw + mode_f * fw
    reads = _dot(rw, m, ((1,), (0,)), _LO)

    out = _split_dot(h, vm_hi_ref[...], vm_lo_ref[...], ((1,), (0,)))
    for r in range(R):
        out = (out +
               _dot(reads[r:r + 1, :], rm_hi_ref[r * W:(r + 1) * W, :],
                    ((1,), (0,)), _LO) +
               _dot(reads[r:r + 1, :], rm_lo_ref[r * W:(r + 1) * W, :],
                    ((1,), (0,)), _LO))
    out_ref[...] = out[None]

    m_ref[...] = m
    u_ref[...] = u
    ww_ref[...] = ww
    l_ref[...] = lnk
    p_ref[...] = prec
    rw_ref[...] = rw
    mn_ref[...] = row_norm2


@jax.jit
def kernel(inputs, W_lstm, b_lstm, IM, VM, RM):
    wx = W_lstm[:X]
    wh = W_lstm[X:]
    wh_hi = wh.astype(jnp.bfloat16)
    wh_lo = (wh - wh_hi.astype(_F32)).astype(jnp.bfloat16)
    b = b_lstm.reshape(1, 4 * X)
    im_p = jnp.pad(IM, ((0, 0), (0, 512 - IF)))
    im_hi = im_p.astype(jnp.bfloat16)
    im_lo = (im_p - im_hi.astype(_F32)).astype(jnp.bfloat16)
    vm_hi = VM.astype(jnp.bfloat16)
    vm_lo = (VM - vm_hi.astype(_F32)).astype(jnp.bfloat16)
    rm_hi = RM.astype(jnp.bfloat16)
    rm_lo = (RM - rm_hi.astype(_F32)).astype(jnp.bfloat16)

    out = pl.pallas_call(
        _dnc_kernel,
        grid=(T,),
        in_specs=[
            pl.BlockSpec((T, X), lambda i: (0, 0)),
            pl.BlockSpec((X, 4 * X), lambda i: (0, 0)),
            pl.BlockSpec((X, 4 * X), lambda i: (0, 0)),
            pl.BlockSpec((X, 4 * X), lambda i: (0, 0)),
            pl.BlockSpec((1, 4 * X), lambda i: (0, 0)),
            pl.BlockSpec((X, 512), lambda i: (0, 0)),
            pl.BlockSpec((X, 512), lambda i: (0, 0)),
            pl.BlockSpec((X, Y), lambda i: (0, 0)),
            pl.BlockSpec((X, Y), lambda i: (0, 0)),
            pl.BlockSpec((R * W, Y), lambda i: (0, 0)),
            pl.BlockSpec((R * W, Y), lambda i: (0, 0)),
        ],
        out_specs=pl.BlockSpec((1, 1, Y), lambda i: (i, 0, 0)),
        out_shape=jax.ShapeDtypeStruct((T, 1, Y), _F32),
        scratch_shapes=[
            pltpu.VMEM((T, 4 * X), _F32),
            pltpu.VMEM((1, X), _F32),
            pltpu.VMEM((1, X), _F32),
            pltpu.VMEM((N, W), _F32),
            pltpu.VMEM((1, N), _F32),
            pltpu.VMEM((1, N), _F32),
            pltpu.VMEM((N, N), _F32),
            pltpu.VMEM((1, N), _F32),
            pltpu.VMEM((R, N), _F32),
            pltpu.VMEM((1, N), _F32),
            pltpu.VMEM((1, 512), _F32),
        ],
    )(inputs, wx, wh_hi, wh_lo, b, im_hi, im_lo, vm_hi, vm_lo, rm_hi, rm_lo)
    return out.reshape(T, Y)


# speedup vs baseline: 1.4223x; 1.0978x over previous
import functools

import jax
import jax.numpy as jnp
from jax.experimental import pallas as pl
from jax.experimental.pallas import tpu as pltpu

T, X, N, W, R, Y = 32, 512, 1024, 64, 4, 128
IF = R * W + 3 * W + 5 * R + 3

_F32 = jnp.float32
_HI = jax.lax.Precision.HIGHEST
_LO = jax.lax.Precision.DEFAULT


def _dot(a, b, dn, prec=_HI):
    return jax.lax.dot_general(a, b, dimension_numbers=(dn, ((), ())),
                               preferred_element_type=_F32, precision=prec)


def _oneplus(x):
    return 1.0 + jnp.maximum(x, 0.0) + jnp.log1p(jnp.exp(-jnp.abs(x)))


def _softmax_cols(x):
    m = jnp.max(x, axis=0, keepdims=True)
    e = jnp.exp(x - m)
    return e / jnp.sum(e, axis=0, keepdims=True)


def _split_dot(a, b_hi, b_lo, dn):
    a_hi = a.astype(jnp.bfloat16)
    a_lo = (a - a_hi.astype(_F32)).astype(jnp.bfloat16)
    return (_dot(a_hi, b_hi, dn, _LO) + _dot(a_hi, b_lo, dn, _LO) +
            _dot(a_lo, b_hi, dn, _LO))


def _lstm_gates(z, c):
    ii = z[:, 0 * X:1 * X]
    ff = z[:, 1 * X:2 * X]
    gg = z[:, 2 * X:3 * X]
    oo = z[:, 3 * X:4 * X]
    c = jax.nn.sigmoid(ff) * c + jax.nn.sigmoid(ii) * jnp.tanh(gg)
    h = jax.nn.sigmoid(oo) * jnp.tanh(c)
    return h, c


def _dnc_kernel(x_ref, wx_ref, wh_hi_ref, wh_lo_ref, b_ref,
                im_hi_ref, im_lo_ref, vm_hi_ref, vm_lo_ref, rm_hi_ref,
                rm_lo_ref,
                out_ref,
                xw_ref, h_ref, c_ref, m_ref, u_ref, ww_ref, l_ref, p_ref,
                rw_ref, mn_ref, iv_ref):
    i = pl.program_id(0)

    @pl.when(i == 0)
    def _init():
        xw = _dot(x_ref[...], wx_ref[...], ((1,), (0,))) + b_ref[...]
        xw_ref[...] = xw
        h0, c0 = _lstm_gates(xw[0:1, :], jnp.zeros((1, X), _F32))
        h_ref[...] = h0
        c_ref[...] = c0
        iv_ref[...] = _split_dot(h0, im_hi_ref[...], im_lo_ref[...],
                                 ((1,), (0,)))
        m_ref[...] = jnp.zeros_like(m_ref)
        u_ref[...] = jnp.zeros_like(u_ref)
        ww_ref[...] = jnp.zeros_like(ww_ref)
        l_ref[...] = jnp.zeros_like(l_ref)
        p_ref[...] = jnp.zeros_like(p_ref)
        rw_ref[...] = jnp.zeros_like(rw_ref)
        mn_ref[...] = jnp.zeros_like(mn_ref)

    h = h_ref[...]
    c = c_ref[...]
    iv = iv_ref[...]

    zn = (xw_ref[pl.ds(jnp.minimum(i + 1, T - 1), 1), :] +
          _split_dot(h, wh_hi_ref[...], wh_lo_ref[...], ((1,), (0,))))
    h_next, c_next = _lstm_gates(zn, c)
    iv_next = _split_dot(h_next, im_hi_ref[...], im_lo_ref[...],
                         ((1,), (0,)))
    h_ref[...] = h_next
    c_ref[...] = c_next
    iv_ref[...] = iv_next

    p = 0
    read_keys = iv[:, p:p + R * W]; p += R * W
    read_strengths = _oneplus(iv[:, p:p + R]); p += R
    write_key = iv[:, p:p + W]; p += W
    write_strength = _oneplus(iv[:, p:p + 1]); p += 1
    erase = jax.nn.sigmoid(iv[:, p:p + W]); p += W
    write_vec = iv[:, p:p + W]; p += W
    free_gates = jax.nn.sigmoid(iv[:, p:p + R]); p += R
    alloc_gate = jax.nn.sigmoid(iv[:, p:p + 1]); p += 1
    write_gate = jax.nn.sigmoid(iv[:, p:p + 1]); p += 1
    rm_base = p

    rw_old = rw_ref[...]
    usage = u_ref[...]
    ww_old = ww_ref[...]

    fg_col = jnp.transpose(free_gates)
    ret_terms = 1.0 - fg_col * rw_old
    retention = (ret_terms[0:1] * ret_terms[1:2] *
                 ret_terms[2:3] * ret_terms[3:4])
    u = retention * (usage + ww_old - usage * ww_old)

    idx_col = jax.lax.broadcasted_iota(jnp.int32, (N, 1), 0)
    idx_row = jax.lax.broadcasted_iota(jnp.int32, (1, N), 1)
    u_col = jnp.transpose(u)
    log_u = jnp.where(u > 0.0, jnp.log(u), -1e5)
    log_u_col = jnp.transpose(log_u)
    log_prod = jnp.sum(
        jnp.where((u_col < u) | ((u_col == u) & (idx_col < idx_row)),
                  log_u_col, 0.0),
        axis=0, keepdims=True)
    alloc = (1.0 - u) * jnp.exp(log_prod)

    m = m_ref[...]
    row_norm = mn_ref[...]
    wk_norm = jnp.sqrt(jnp.sum(write_key * write_key, axis=1, keepdims=True))
    mk = _dot(write_key, m, ((1,), (1,)), _LO)
    wlog = write_strength * mk / (row_norm * wk_norm + 1e-8)
    wcw = jax.nn.softmax(wlog, axis=1)

    ww = write_gate * (alloc_gate * alloc + (1.0 - alloc_gate) * wcw)

    ww_col = jnp.transpose(ww)
    m = m * (1.0 - ww_col * erase) + ww_col * write_vec

    prec = p_ref[...] * (1.0 - jnp.sum(ww)) + ww
    NB = 4
    BL = N // NB
    fw = None
    bw_parts = []
    for bidx in range(NB):
        sl = pl.ds(bidx * BL, BL)
        wwc = ww_col[bidx * BL:(bidx + 1) * BL, :]
        lnk_b = l_ref[sl, :] * (1.0 - wwc - ww) + wwc * prec
        l_ref[sl, :] = lnk_b
        fpart = _dot(rw_old[:, bidx * BL:(bidx + 1) * BL], lnk_b,
                     ((1,), (0,)), _LO)
        fw = fpart if fw is None else fw + fpart
        bw_parts.append(_dot(rw_old, lnk_b, ((1,), (1,)), _LO))
    bw = jnp.concatenate(bw_parts, axis=1)

    keys = jnp.concatenate(
        [read_keys[:, r * W:(r + 1) * W] for r in range(R)], axis=0)
    k_norm = jnp.sqrt(jnp.sum(keys * keys, axis=1, keepdims=True))
    row_norm2 = jnp.sqrt(_dot(jnp.ones((1, W), _F32), m * m,
                              ((1,), (1,)), _LO))
    mkr = _dot(keys, m, ((1,), (1,)), _LO)
    cos = mkr / (k_norm * row_norm2 + 1e-8)
    rs_col = jnp.transpose(read_strengths)
    rcw = jax.nn.softmax(rs_col * cos, axis=1)

    mode_b, mode_c, mode_f = [], [], []
    for r in range(R):
        lg = iv[:, rm_base + 3 * r:rm_base + 3 * (r + 1)]
        e = jnp.exp(lg - jnp.max(lg, axis=1, keepdims=True))
        sm = e / jnp.sum(e, axis=1, keepdims=True)
        mode_b.append(sm[:, 0:1])
        mode_c.append(sm[:, 1:2])
        mode_f.append(sm[:, 2:3])
    mode_b = jnp.concatenate(mode_b, axis=0)
    mode_c = jnp.concatenate(mode_c, axis=0)
    mode_f = jnp.concatenate(mode_f, axis=0)

    rw = mode_b * bw + mode_c * rcw + mode_f * fw
    reads = _dot(rw, m, ((1,), (0,)), _LO)

    out = _split_dot(h, vm_hi_ref[...], vm_lo_ref[...], ((1,), (0,)))
    for r in range(R):
        out = (out +
               _dot(reads[r:r + 1, :], rm_hi_ref[r * W:(r + 1) * W, :],
                    ((1,), (0,)), _LO) +
               _dot(reads[r:r + 1, :], rm_lo_ref[r * W:(r + 1) * W, :],
                    ((1,), (0,)), _LO))
    out_ref[...] = out[None]

    m_ref[...] = m
    u_ref[...] = u
    ww_ref[...] = ww
    p_ref[...] = prec
    rw_ref[...] = rw
    mn_ref[...] = row_norm2


@jax.jit
def kernel(inputs, W_lstm, b_lstm, IM, VM, RM):
    wx = W_lstm[:X]
    wh = W_lstm[X:]
    wh_hi = wh.astype(jnp.bfloat16)
    wh_lo = (wh - wh_hi.astype(_F32)).astype(jnp.bfloat16)
    b = b_lstm.reshape(1, 4 * X)
    im_p = jnp.pad(IM, ((0, 0), (0, 512 - IF)))
    im_hi = im_p.astype(jnp.bfloat16)
    im_lo = (im_p - im_hi.astype(_F32)).astype(jnp.bfloat16)
    vm_hi = VM.astype(jnp.bfloat16)
    vm_lo = (VM - vm_hi.astype(_F32)).astype(jnp.bfloat16)
    rm_hi = RM.astype(jnp.bfloat16)
    rm_lo = (RM - rm_hi.astype(_F32)).astype(jnp.bfloat16)

    out = pl.pallas_call(
        _dnc_kernel,
        grid=(T,),
        in_specs=[
            pl.BlockSpec((T, X), lambda i: (0, 0)),
            pl.BlockSpec((X, 4 * X), lambda i: (0, 0)),
            pl.BlockSpec((X, 4 * X), lambda i: (0, 0)),
            pl.BlockSpec((X, 4 * X), lambda i: (0, 0)),
            pl.BlockSpec((1, 4 * X), lambda i: (0, 0)),
            pl.BlockSpec((X, 512), lambda i: (0, 0)),
            pl.BlockSpec((X, 512), lambda i: (0, 0)),
            pl.BlockSpec((X, Y), lambda i: (0, 0)),
            pl.BlockSpec((X, Y), lambda i: (0, 0)),
            pl.BlockSpec((R * W, Y), lambda i: (0, 0)),
            pl.BlockSpec((R * W, Y), lambda i: (0, 0)),
        ],
        out_specs=pl.BlockSpec((1, 1, Y), lambda i: (i, 0, 0)),
        out_shape=jax.ShapeDtypeStruct((T, 1, Y), _F32),
        scratch_shapes=[
            pltpu.VMEM((T, 4 * X), _F32),
            pltpu.VMEM((1, X), _F32),
            pltpu.VMEM((1, X), _F32),
            pltpu.VMEM((N, W), _F32),
            pltpu.VMEM((1, N), _F32),
            pltpu.VMEM((1, N), _F32),
            pltpu.VMEM((N, N), _F32),
            pltpu.VMEM((1, N), _F32),
            pltpu.VMEM((R, N), _F32),
            pltpu.VMEM((1, N), _F32),
            pltpu.VMEM((1, 512), _F32),
        ],
    )(inputs, wx, wh_hi, wh_lo, b, im_hi, im_lo, vm_hi, vm_lo, rm_hi, rm_lo)
    return out.reshape(T, Y)
